# Initial kernel scaffold; baseline (speedup 1.0000x reference)
import jax
import jax.numpy as jnp
from jax.experimental import pallas as pl


def kernel(x, router_w, router_b, w1, w2):
    raise NotImplementedError("write your pallas kernel here")


# speedup vs baseline: 6.1939x; 6.1939x over previous
import functools

import jax
import jax.numpy as jnp
from jax.experimental import pallas as pl
from jax.experimental.pallas import tpu as pltpu

NUM_EXPERTS = 8
TOP_K = 2


def _moe_kernel(x_ref, rw_ref, rb_ref, w1_ref, w2_ref, out_ref, w_scratch):
    e = pl.program_id(0)

    @pl.when(e == 0)
    def _router():
        x = x_ref[...]
        logits = jax.lax.dot_general(
            x, rw_ref[...],
            dimension_numbers=(((1,), (1,)), ((), ())),
            preferred_element_type=jnp.float32,
        ) + rb_ref[...]
        m = jnp.max(logits, axis=-1, keepdims=True)
        ex = jnp.exp(logits - m)
        p = ex / jnp.sum(ex, axis=-1, keepdims=True)
        iota = jax.lax.broadcasted_iota(jnp.int32, p.shape, 1)
        m1 = jnp.max(p, axis=-1, keepdims=True)
        i1 = jnp.min(jnp.where(p == m1, iota, NUM_EXPERTS), axis=-1, keepdims=True)
        mask1 = iota == i1
        p2 = jnp.where(mask1, -jnp.inf, p)
        m2 = jnp.max(p2, axis=-1, keepdims=True)
        i2 = jnp.min(jnp.where(p2 == m2, iota, NUM_EXPERTS), axis=-1, keepdims=True)
        mask2 = iota == i2
        denom = m1 + m2
        w_scratch[...] = jnp.where(mask1 | mask2, p / denom, 0.0)

    @pl.when(e == 0)
    def _zero():
        out_ref[...] = jnp.zeros_like(out_ref)

    x = x_ref[...]
    h = jnp.dot(x, w1_ref[0], preferred_element_type=jnp.float32)
    h = 0.5 * h * (1.0 + jax.lax.erf(h * 0.7071067811865476))
    y = jnp.dot(h, w2_ref[0], preferred_element_type=jnp.float32)
    w_all = w_scratch[...]
    eiota = jax.lax.broadcasted_iota(jnp.int32, w_all.shape, 1)
    wcol = jnp.sum(jnp.where(eiota == e, w_all, 0.0), axis=-1, keepdims=True)
    out_ref[...] += wcol * y


def kernel(x, router_w, router_b, w1, w2):
    B, T, H = x.shape
    N = B * T
    F = w1.shape[-1]
    x_flat = x.reshape(N, H)
    rb = router_b.reshape(1, NUM_EXPERTS)

    out = pl.pallas_call(
        _moe_kernel,
        grid=(NUM_EXPERTS,),
        in_specs=[
            pl.BlockSpec((N, H), lambda e: (0, 0)),
            pl.BlockSpec((NUM_EXPERTS, H), lambda e: (0, 0)),
            pl.BlockSpec((1, NUM_EXPERTS), lambda e: (0, 0)),
            pl.BlockSpec((1, H, F), lambda e: (e, 0, 0)),
            pl.BlockSpec((1, F, H), lambda e: (e, 0, 0)),
        ],
        out_specs=pl.BlockSpec((N, H), lambda e: (0, 0)),
        out_shape=jax.ShapeDtypeStruct((N, H), jnp.float32),
        scratch_shapes=[pltpu.VMEM((N, NUM_EXPERTS), jnp.float32)],
    )(x_flat, router_w, rb, w1, w2)
    return out.reshape(B, T, H)


# speedup vs baseline: 6.1955x; 1.0003x over previous
import functools

import jax
import jax.numpy as jnp
from jax.experimental import pallas as pl
from jax.experimental.pallas import tpu as pltpu

NUM_EXPERTS = 8
TOP_K = 2


def _moe_kernel(x_ref, rw_ref, rb_ref, w1_ref, w2_ref, out_ref, w_scratch):
    e = pl.program_id(0)

    @pl.when(e == 0)
    def _router():
        x = x_ref[...]
        logits = jax.lax.dot_general(
            x, rw_ref[...],
            dimension_numbers=(((1,), (1,)), ((), ())),
            preferred_element_type=jnp.float32,
        ) + rb_ref[...]
        m = jnp.max(logits, axis=-1, keepdims=True)
        ex = jnp.exp(logits - m)
        p = ex / jnp.sum(ex, axis=-1, keepdims=True)
        iota = jax.lax.broadcasted_iota(jnp.int32, p.shape, 1)
        m1 = jnp.max(p, axis=-1, keepdims=True)
        i1 = jnp.min(jnp.where(p == m1, iota, NUM_EXPERTS), axis=-1, keepdims=True)
        mask1 = iota == i1
        p2 = jnp.where(mask1, -jnp.inf, p)
        m2 = jnp.max(p2, axis=-1, keepdims=True)
        i2 = jnp.min(jnp.where(p2 == m2, iota, NUM_EXPERTS), axis=-1, keepdims=True)
        mask2 = iota == i2
        denom = m1 + m2
        w_scratch[...] = jnp.where(mask1 | mask2, p / denom, 0.0)

    @pl.when(e == 0)
    def _zero():
        out_ref[...] = jnp.zeros_like(out_ref)

    x = x_ref[...].astype(jnp.bfloat16)
    h = jnp.dot(x, w1_ref[0].astype(jnp.bfloat16),
                preferred_element_type=jnp.float32)
    h = 0.5 * h * (1.0 + jax.lax.erf(h * 0.7071067811865476))
    y = jnp.dot(h.astype(jnp.bfloat16), w2_ref[0].astype(jnp.bfloat16),
                preferred_element_type=jnp.float32)
    w_all = w_scratch[...]
    eiota = jax.lax.broadcasted_iota(jnp.int32, w_all.shape, 1)
    wcol = jnp.sum(jnp.where(eiota == e, w_all, 0.0), axis=-1, keepdims=True)
    out_ref[...] += wcol * y


def kernel(x, router_w, router_b, w1, w2):
    B, T, H = x.shape
    N = B * T
    F = w1.shape[-1]
    x_flat = x.reshape(N, H)
    rb = router_b.reshape(1, NUM_EXPERTS)

    out = pl.pallas_call(
        _moe_kernel,
        grid=(NUM_EXPERTS,),
        in_specs=[
            pl.BlockSpec((N, H), lambda e: (0, 0)),
            pl.BlockSpec((NUM_EXPERTS, H), lambda e: (0, 0)),
            pl.BlockSpec((1, NUM_EXPERTS), lambda e: (0, 0)),
            pl.BlockSpec((1, H, F), lambda e: (e, 0, 0)),
            pl.BlockSpec((1, F, H), lambda e: (e, 0, 0)),
        ],
        out_specs=pl.BlockSpec((N, H), lambda e: (0, 0)),
        out_shape=jax.ShapeDtypeStruct((N, H), jnp.float32),
        scratch_shapes=[pltpu.VMEM((N, NUM_EXPERTS), jnp.float32)],
    )(x_flat, router_w, rb, w1, w2)
    return out.reshape(B, T, H)
